# Initial kernel scaffold; baseline (speedup 1.0000x reference)
import jax
import jax.numpy as jnp
from jax.experimental import pallas as pl


def kernel(uv, x, upsample, dw0, pw0, dw1, pw1, dw2, pw2, dw3, pw3, dwh, pwh, sp0, sp1, sp2, sp3, row0, col0, val0, row1, col1, val1, row2, col2, val2, row3, col3, val3):
    raise NotImplementedError("write your pallas kernel here")


# speedup vs baseline: 16.5401x; 16.5401x over previous
import functools

import jax
import jax.numpy as jnp
from jax import lax
from jax.experimental import pallas as pl
from jax.experimental.pallas import tpu as pltpu
from jax.experimental.pallas import tpu_sc as plsc

_L = 16


def _entry_body(uv_ref, x_ref, up_ref, out_ref):
    B = uv_ref.shape[0]
    P = uv_ref.shape[1]
    HW = x_ref.shape[2]
    q = lax.broadcasted_iota(jnp.int32, (P, HW), 1)
    for b in range(B):
        uvb = uv_ref[b]
        g = jnp.clip((uvb - 0.5) * 2.0, -1.0, 1.0)
        gx = (g[:, 0:1] + 1.0) * 31.5
        gy = (g[:, 1:2] + 1.0) * 31.5
        x0 = jnp.floor(gx)
        y0 = jnp.floor(gy)
        wx1 = gx - x0
        wy1 = gy - y0
        x0i = jnp.clip(x0, 0.0, 63.0).astype(jnp.int32)
        x1i = jnp.clip(x0 + 1.0, 0.0, 63.0).astype(jnp.int32)
        y0i = jnp.clip(y0, 0.0, 63.0).astype(jnp.int32)
        y1i = jnp.clip(y0 + 1.0, 0.0, 63.0).astype(jnp.int32)

        def tap(yi, xi, w):
            return jnp.where(q == yi * 64 + xi, w, 0.0)

        S = (tap(y0i, x0i, (1.0 - wy1) * (1.0 - wx1))
             + tap(y0i, x1i, (1.0 - wy1) * wx1)
             + tap(y1i, x0i, wy1 * (1.0 - wx1))
             + tap(y1i, x1i, wy1 * wx1))
        featT = lax.dot_general(S, x_ref[b], (((1,), (1,)), ((), ())),
                                preferred_element_type=jnp.float32)
        out_ref[:, b, :] = jnp.dot(up_ref[...], featT,
                                   preferred_element_type=jnp.float32)


def _entry(uv, xflat, up):
    B, P, _ = uv.shape
    C = xflat.shape[1]
    HW = xflat.shape[2]
    V4 = up.shape[0]
    return pl.pallas_call(
        _entry_body,
        in_specs=[
            pl.BlockSpec((B, P, 2), lambda: (0, 0, 0)),
            pl.BlockSpec((B, C, HW), lambda: (0, 0, 0)),
            pl.BlockSpec((V4, P), lambda: (0, 0)),
        ],
        out_specs=pl.BlockSpec((V4, B, C), lambda: (0, 0, 0)),
        out_shape=jax.ShapeDtypeStruct((V4, B, C), jnp.float32),
    )(uv, xflat, up)


def _mm_body(a_ref, b_ref, o_ref, *, relu):
    r = jnp.dot(a_ref[...], b_ref[...], preferred_element_type=jnp.float32)
    o_ref[...] = jnp.maximum(r, 0.0) if relu else r


def _mm(a, bT, relu, bm=2048):
    M, K = a.shape
    N = bT.shape[1]
    return pl.pallas_call(
        functools.partial(_mm_body, relu=relu),
        grid=(M // bm,),
        in_specs=[
            pl.BlockSpec((bm, K), lambda i: (i, 0)),
            pl.BlockSpec((K, N), lambda i: (0, 0)),
        ],
        out_specs=pl.BlockSpec((bm, N), lambda i: (i, 0)),
        out_shape=jax.ShapeDtypeStruct((M, N), jnp.float32),
    )(a, bT)


def _sc_stage(table, idxflat, w, K, D, Vout, mode):
    info = plsc.get_sparse_core_info()
    NC, NS = info.num_cores, info.num_subcores
    NW = NC * NS
    per_w = Vout // NW
    if K == 9:
        n = 8
    else:
        n = 16 if D >= 1024 else 32
    nchunks = per_w // n
    nlanes = D // _L

    if mode == "spiral":
        w_scratch = pltpu.VMEM((K, D), jnp.float32)
    else:
        w_scratch = pltpu.VMEM((n * K + _L,), jnp.float32)

    @functools.partial(
        pl.kernel,
        out_type=jax.ShapeDtypeStruct((Vout, D), jnp.float32),
        mesh=plsc.VectorSubcoreMesh(core_axis_name="c", subcore_axis_name="s"),
        scratch_types=[
            pltpu.VMEM((n * K,), jnp.int32),
            pltpu.VMEM((n * K, D), jnp.float32),
            pltpu.VMEM((n, D), jnp.float32),
            w_scratch,
            pltpu.SemaphoreType.DMA,
        ],
    )
    def k(table_hbm, idx_hbm, w_hbm, out_hbm, idx_v, buf, acc, wv, sem):
        wid = lax.axis_index("s") * NC + lax.axis_index("c")
        base0 = wid * per_w
        if mode == "spiral":
            pltpu.sync_copy(w_hbm, wv)

        def chunk_body(ci, carry):
            base = base0 + ci * n
            pltpu.sync_copy(idx_hbm.at[pl.ds(base * K, n * K)], idx_v)
            if mode == "pool":
                pltpu.sync_copy(w_hbm.at[pl.ds(base * K, n * K)],
                                wv.at[pl.ds(0, n * K)])
            pltpu.async_copy(table_hbm.at[idx_v], buf, sem).wait()

            if mode == "spiral":
                def lane_body(j, c2):
                    sl = pl.ds(j * _L, _L)
                    wregs = [wv[s, sl] for s in range(K)]
                    for i in range(n):
                        a = wregs[0] * buf[i * K, sl]
                        for s in range(1, K):
                            a = a + wregs[s] * buf[i * K + s, sl]
                        acc[i, sl] = a
                    return c2
                lax.fori_loop(0, nlanes, lane_body, 0)
            else:
                for i in range(n):
                    wvec = wv[pl.ds(i * K, _L)]
                    wregs = [wvec[s] for s in range(K)]

                    def lane_body(j, c2, i=i, wregs=wregs):
                        sl = pl.ds(j * _L, _L)
                        a = wregs[0] * buf[i * K, sl]
                        for s in range(1, K):
                            a = a + wregs[s] * buf[i * K + s, sl]
                        acc[i, sl] = a
                        return c2
                    lax.fori_loop(0, nlanes, lane_body, 0)

            pltpu.sync_copy(acc, out_hbm.at[pl.ds(base, n)])
            return carry

        lax.fori_loop(0, nchunks, chunk_body, 0)

    return k(table, idxflat, w)


def kernel(uv, x, upsample, dw0, pw0, dw1, pw1, dw2, pw2, dw3, pw3, dwh, pwh,
           sp0, sp1, sp2, sp3,
           row0, col0, val0, row1, col1, val1, row2, col2, val2,
           row3, col3, val3):
    B, C0 = x.shape[0], x.shape[1]
    xflat = x.reshape(B, C0, x.shape[2] * x.shape[3])
    h = _entry(uv, xflat, upsample)

    levels = [
        (col3, val3, sp3, dw0, pw0),
        (col2, val2, sp2, dw1, pw1),
        (col1, val1, sp1, dw2, pw2),
        (col0, val0, sp0, dw3, pw3),
    ]
    for col, val, sp, dwl, pwl in levels:
        Vin, Bq, Cin = h.shape
        D = Bq * Cin
        Vout = col.shape[0] // 3
        pooled = _sc_stage(h.reshape(Vin, D), col, val, 3, D, Vout, "pool")
        dwt = jnp.tile(dwl.T[:, None, :], (1, Bq, 1)).reshape(sp.shape[1], D)
        g = _sc_stage(pooled, sp.reshape(-1), dwt, sp.shape[1], D, Vout,
                      "spiral")
        Cout = pwl.shape[0]
        o = _mm(g.reshape(Vout * Bq, Cin), pwl.T, relu=True)
        h = o.reshape(Vout, Bq, Cout)

    Vin, Bq, Cin = h.shape
    D = Bq * Cin
    dwt = jnp.tile(dwh.T[:, None, :], (1, Bq, 1)).reshape(sp0.shape[1], D)
    gh = _sc_stage(h.reshape(Vin, D), sp0.reshape(-1), dwt, sp0.shape[1], D,
                   Vin, "spiral")
    pred = _mm(gh.reshape(Vin * Bq, Cin), pwh.T, relu=False)
    return pred.reshape(Vin, Bq, 3).transpose(1, 0, 2)


# speedup vs baseline: 25.6371x; 1.5500x over previous
import functools

import jax
import jax.numpy as jnp
from jax import lax
from jax.experimental import pallas as pl
from jax.experimental.pallas import tpu as pltpu
from jax.experimental.pallas import tpu_sc as plsc

_L = 16


def _entry_body(uv_ref, x_ref, up_ref, out0_ref, out1_ref):
    B = uv_ref.shape[0]
    P = uv_ref.shape[1]
    C = x_ref.shape[1]
    HW = x_ref.shape[2]
    q = lax.broadcasted_iota(jnp.int32, (P, HW), 1)
    outs = (out0_ref, out1_ref)
    for b in range(B):
        uvb = uv_ref[b]
        g = jnp.clip((uvb - 0.5) * 2.0, -1.0, 1.0)
        gx = (g[:, 0:1] + 1.0) * 31.5
        gy = (g[:, 1:2] + 1.0) * 31.5
        x0 = jnp.floor(gx)
        y0 = jnp.floor(gy)
        wx1 = gx - x0
        wy1 = gy - y0
        x0i = jnp.clip(x0, 0.0, 63.0).astype(jnp.int32)
        x1i = jnp.clip(x0 + 1.0, 0.0, 63.0).astype(jnp.int32)
        y0i = jnp.clip(y0, 0.0, 63.0).astype(jnp.int32)
        y1i = jnp.clip(y0 + 1.0, 0.0, 63.0).astype(jnp.int32)

        def tap(yi, xi, w):
            return jnp.where(q == yi * 64 + xi, w, 0.0)

        S = (tap(y0i, x0i, (1.0 - wy1) * (1.0 - wx1))
             + tap(y0i, x1i, (1.0 - wy1) * wx1)
             + tap(y1i, x0i, wy1 * (1.0 - wx1))
             + tap(y1i, x1i, wy1 * wx1))
        featT = lax.dot_general(S, x_ref[b], (((1,), (1,)), ((), ())),
                                preferred_element_type=jnp.float32)
        m = jnp.dot(up_ref[...], featT, preferred_element_type=jnp.float32)
        outs[b // 2][:, pl.ds((b % 2) * C, C)] = m


def _entry(uv, xflat, up):
    B, P, _ = uv.shape
    C = xflat.shape[1]
    HW = xflat.shape[2]
    V4 = up.shape[0]
    return pl.pallas_call(
        _entry_body,
        in_specs=[
            pl.BlockSpec((B, P, 2), lambda: (0, 0, 0)),
            pl.BlockSpec((B, C, HW), lambda: (0, 0, 0)),
            pl.BlockSpec((V4, P), lambda: (0, 0)),
        ],
        out_specs=[pl.BlockSpec((V4, 2 * C), lambda: (0, 0))] * 2,
        out_shape=[jax.ShapeDtypeStruct((V4, 2 * C), jnp.float32)] * 2,
    )(uv, xflat, up)


def _mm_body(a_ref, b_ref, o_ref, *, relu):
    r = jnp.dot(a_ref[...], b_ref[...], preferred_element_type=jnp.float32)
    o_ref[...] = jnp.maximum(r, 0.0) if relu else r


def _mm(a, bT, relu, bm=2048):
    M, K = a.shape
    N = bT.shape[1]
    return pl.pallas_call(
        functools.partial(_mm_body, relu=relu),
        grid=(M // bm,),
        in_specs=[
            pl.BlockSpec((bm, K), lambda i: (i, 0)),
            pl.BlockSpec((K, N), lambda i: (0, 0)),
        ],
        out_specs=pl.BlockSpec((bm, N), lambda i: (i, 0)),
        out_shape=jax.ShapeDtypeStruct((M, N), jnp.float32),
    )(a, bT)


_VSUB = {3: 16, 9: 8}
_CHUNK = {
    (9, 512): 8, (9, 256): 16, (9, 128): 32, (9, 64): 64,
    (3, 512): 16, (3, 256): 48, (3, 128): 96,
}


def _sc_stage(table, idxflat, w, K, D, Vout, mode):
    info = plsc.get_sparse_core_info()
    NC, NS = info.num_cores, info.num_subcores
    NW = NC * NS
    per_w = Vout // NW
    n = _CHUNK[(K, D)]
    vsub = _VSUB[K]
    nsub = n // vsub
    nchunks = per_w // n
    assert nchunks % 2 == 0 and per_w % n == 0 and n % vsub == 0
    nlanes = D // _L

    if mode == "spiral":
        w_scratch = pltpu.VMEM((K, D), jnp.float32)
    else:
        w_scratch = pltpu.VMEM((per_w * K + _L,), jnp.float32)

    @functools.partial(
        pl.kernel,
        out_type=jax.ShapeDtypeStruct((Vout, D), jnp.float32),
        mesh=plsc.VectorSubcoreMesh(core_axis_name="c", subcore_axis_name="s"),
        scratch_types=[
            pltpu.VMEM((per_w * K,), jnp.int32),
            pltpu.VMEM((n * K, D), jnp.float32),
            pltpu.VMEM((n * K, D), jnp.float32),
            pltpu.VMEM((n, D), jnp.float32),
            pltpu.VMEM((n, D), jnp.float32),
            w_scratch,
            pltpu.SemaphoreType.DMA,
            pltpu.SemaphoreType.DMA,
            pltpu.SemaphoreType.DMA,
            pltpu.SemaphoreType.DMA,
        ],
    )
    def k(table_hbm, idx_hbm, w_hbm, out_hbm, idx_all, buf0, buf1,
          acc0, acc1, wv, semg0, semg1, sems0, sems1):
        bufs = (buf0, buf1)
        accs = (acc0, acc1)
        semg = (semg0, semg1)
        sems = (sems0, sems1)
        wid = lax.axis_index("s") * NC + lax.axis_index("c")
        base0 = wid * per_w
        pltpu.sync_copy(idx_hbm.at[pl.ds(base0 * K, per_w * K)], idx_all)
        if mode == "spiral":
            pltpu.sync_copy(w_hbm, wv)
        else:
            pltpu.sync_copy(w_hbm.at[pl.ds(base0 * K, per_w * K)],
                            wv.at[pl.ds(0, per_w * K)])

        def fire(b, ci):
            for s in range(nsub):
                off = ci * n * K + s * vsub * K
                pltpu.async_copy(
                    table_hbm.at[idx_all.at[pl.ds(off, vsub * K)]],
                    bufs[b].at[pl.ds(s * vsub * K, vsub * K)],
                    semg[b])

        def drain_gather(b):
            pltpu.make_async_copy(out_hbm.at[pl.ds(0, n * K)], bufs[b],
                                  semg[b]).wait()

        def drain_store(b):
            pltpu.make_async_copy(accs[b], out_hbm.at[pl.ds(base0, n)],
                                  sems[b]).wait()

        def compute(b, ci):
            buf = bufs[b]
            acc = accs[b]
            if mode == "spiral":
                def lane_body(j, c2):
                    sl = pl.ds(j * _L, _L)
                    wregs = [wv[s, sl] for s in range(K)]
                    for i in range(n):
                        a = wregs[0] * buf[i * K, sl]
                        for s in range(1, K):
                            a = a + wregs[s] * buf[i * K + s, sl]
                        acc[i, sl] = a
                    return c2
                lax.fori_loop(0, nlanes, lane_body, 0)
            else:
                def row_body(i, c2):
                    wvec = wv[pl.ds((ci * n + i) * K, _L)]
                    wregs = [wvec[s] for s in range(K)]
                    for j in range(nlanes):
                        sl = pl.ds(j * _L, _L)
                        a = wregs[0] * buf[i * K, sl]
                        for s in range(1, K):
                            a = a + wregs[s] * buf[i * K + s, sl]
                        acc[i, sl] = a
                    return c2
                lax.fori_loop(0, n, row_body, 0)

        fire(0, 0)

        def outer(cc, carry):
            for b in range(2):
                ci = cc * 2 + b
                nb = 1 - b

                @pl.when(ci + 1 < nchunks)
                def _():
                    fire(nb, ci + 1)

                drain_gather(b)

                @pl.when(ci >= 2)
                def _():
                    drain_store(b)

                compute(b, ci)
                pltpu.async_copy(accs[b],
                                 out_hbm.at[pl.ds(base0 + ci * n, n)],
                                 sems[b])
            return carry

        lax.fori_loop(0, nchunks // 2, outer, 0)
        drain_store(0)
        drain_store(1)

    return k(table, idxflat, w)


def kernel(uv, x, upsample, dw0, pw0, dw1, pw1, dw2, pw2, dw3, pw3, dwh, pwh,
           sp0, sp1, sp2, sp3,
           row0, col0, val0, row1, col1, val1, row2, col2, val2,
           row3, col3, val3):
    B, C0 = x.shape[0], x.shape[1]
    xflat = x.reshape(B, C0, x.shape[2] * x.shape[3])
    hs = list(_entry(uv, xflat, upsample))

    levels = [
        (col3, val3, sp3, dw0, pw0),
        (col2, val2, sp2, dw1, pw1),
        (col1, val1, sp1, dw2, pw2),
        (col0, val0, sp0, dw3, pw3),
    ]
    for col, val, sp, dwl, pwl in levels:
        Cin = dwl.shape[0]
        D = 2 * Cin
        Vout = col.shape[0] // 3
        Cout = pwl.shape[0]
        dwt = jnp.tile(dwl.T[:, None, :], (1, 2, 1)).reshape(sp.shape[1], D)
        for g in range(2):
            pooled = _sc_stage(hs[g], col, val, 3, D, Vout, "pool")
            gg = _sc_stage(pooled, sp.reshape(-1), dwt, sp.shape[1], D, Vout,
                           "spiral")
            o = _mm(gg.reshape(Vout * 2, Cin), pwl.T, relu=True)
            hs[g] = o.reshape(Vout, 2 * Cout)

    Cin = dwh.shape[0]
    D = B * Cin
    V0 = sp0.shape[0]
    merged = jnp.concatenate(hs, axis=1)
    dwt = jnp.tile(dwh.T[:, None, :], (1, B, 1)).reshape(sp0.shape[1], D)
    gh = _sc_stage(merged, sp0.reshape(-1), dwt, sp0.shape[1], D, V0,
                   "spiral")
    pred = _mm(gh.reshape(V0 * B, Cin), pwh.T, relu=False)
    return pred.reshape(V0, B, 3).transpose(1, 0, 2)
